# baseline (speedup 1.0000x reference)
import functools

import jax
import jax.numpy as jnp
from jax import lax
from jax.experimental import pallas as pl
from jax.experimental.pallas import tpu as pltpu
from jax.experimental.pallas import tpu_sc as plsc

_NC, _NS = 2, 16
_NW = _NC * _NS
_EB = 128
_ZR = 64
_F32 = jnp.float32
_SPW = 2_060_000


def _rup(v, m):
    return (v + m - 1) // m * m


def _cap(n_out, k):
    n_acc = _rup(n_out, 128) + 128
    fixed = 16 * 2 * 8 * _EB
    per_ccw = n_acc + 16 * (8 * _EB + _ZR)
    return max(16, (_SPW - fixed) // per_ccw // 16 * 16)


@functools.lru_cache(maxsize=None)
def _na_fn(n_src, n_out, n_chunks, ccw, k):
    n_op = _rup(n_out, 128)
    n_acc = n_op + 128
    stripe = n_acc // _NS
    rows_out = n_op // _NS
    nfull, rem = divmod(stripe, _ZR)
    mesh = plsc.VectorSubcoreMesh(core_axis_name="c", subcore_axis_name="s")

    nw = k // 4

    def body(x_h, srcr_h, dstr_h, out_h, acc_sh, idxs_b, idxd_b, rows_b,
             zb_v, si0, si1, *sg):
        ci = lax.axis_index("c")
        si = lax.axis_index("s")
        wid = ci * _NS + si
        myis = srcr_h.at[wid]
        myid = dstr_h.at[wid]

        def _z(r, carry):
            for t in range(ccw // 16):
                zb_v[r, pl.ds(t * 16, 16)] = jnp.zeros((16,), _F32)
            return carry
        lax.fori_loop(0, _ZR, _z, 0)

        zbase = si * stripe
        obase = si * rows_out

        def ifetch(w, q, sem):
            pltpu.async_copy(myis.at[pl.ds(4 * w, 4)],
                             idxs_b.at[pl.ds(4 * q, 4)], sem)
            pltpu.async_copy(myid.at[pl.ds(4 * w, 4)],
                             idxd_b.at[pl.ds(4 * q, 4)], sem)

        def iwait(q, sem):
            pltpu.make_async_copy(myis.at[pl.ds(0, 4)],
                                  idxs_b.at[pl.ds(4 * q, 4)], sem).wait()
            pltpu.make_async_copy(myid.at[pl.ds(0, 4)],
                                  idxd_b.at[pl.ds(4 * q, 4)], sem).wait()

        def _chunk(cc, carry):
            def _zc(t, c2):
                pltpu.sync_copy(zb_v, acc_sh.at[pl.ds(zbase + t * _ZR, _ZR)])
                return c2
            lax.fori_loop(0, nfull, _zc, 0)
            if rem:
                pltpu.sync_copy(zb_v.at[pl.ds(0, rem)],
                                acc_sh.at[pl.ds(zbase + nfull * _ZR, rem)])
            plsc.subcore_barrier()

            xcc = x_h.at[cc]

            def gissue(q, r):
                pltpu.async_copy(xcc.at[idxs_b.at[4 * q + r]],
                                 rows_b.at[4 * q + r], sg[4 * q + r])

            def gdrain(q, r):
                pltpu.make_async_copy(xcc.at[pl.ds(0, _EB)],
                                      rows_b.at[4 * q + r],
                                      sg[4 * q + r]).wait()

            def scat(q, r):
                pltpu.sync_copy(rows_b.at[4 * q + r],
                                acc_sh.at[idxd_b.at[4 * q + r]], add=True)

            ifetch(0, 0, si0)
            ifetch(1, 1, si1)
            iwait(0, si0)
            for r in range(4):
                gissue(0, r)

            def _ww(ww, carry2):
                w0 = 2 * ww
                iwait(1, si1)
                for r in range(4):
                    gissue(1, r)
                for r in range(4):
                    gdrain(0, r)
                    scat(0, r)

                @pl.when(w0 + 2 < nw)
                def _():
                    ifetch(w0 + 2, 0, si0)

                w1 = w0 + 1

                @pl.when(w1 + 1 < nw)
                def _():
                    iwait(0, si0)
                    for r in range(4):
                        gissue(0, r)
                for r in range(4):
                    gdrain(1, r)
                    scat(1, r)

                @pl.when(w1 + 2 < nw)
                def _():
                    ifetch(w1 + 2, 1, si1)
                return carry2
            lax.fori_loop(0, nw // 2, _ww, 0)
            plsc.subcore_barrier()

            pltpu.sync_copy(acc_sh.at[pl.ds(obase, rows_out)],
                            out_h.at[cc, ci, pl.ds(obase, rows_out)])
            plsc.subcore_barrier()
            return carry
        lax.fori_loop(0, n_chunks, _chunk, 0)

    return pl.kernel(
        body,
        out_type=jax.ShapeDtypeStruct((n_chunks, _NC, n_op, ccw), _F32),
        mesh=mesh,
        compiler_params=pltpu.CompilerParams(use_tc_tiling_on_sc=False),
        scratch_types=[
            pltpu.VMEM_SHARED((n_acc, ccw), _F32),
            pltpu.VMEM((8, _EB), jnp.int32),
            pltpu.VMEM((8, _EB), jnp.int32),
            pltpu.VMEM((8, _EB, ccw), _F32),
            pltpu.VMEM((_ZR, ccw), _F32),
        ] + [pltpu.SemaphoreType.DMA] * 10,
    )


def _graph(src, dst, n_out):
    e = src.shape[0]
    epad = _rup(e, _NW * _EB * 8)
    k = epad // (_NW * _EB)
    srcr = jnp.pad(src, (0, epad - e)).reshape(_NW, k, _EB)
    dstr = jnp.pad(dst, (0, epad - e),
                   constant_values=n_out).reshape(_NW, k, _EB)
    return (srcr, dstr, k, n_out)


def _na_multi(xs, g):
    srcr, dstr, k, n_out = g
    n = xs[0].shape[0]
    widths = [x.shape[1] for x in xs]
    offs = [0]
    for w in widths:
        offs.append(offs[-1] + w)
    c = offs[-1]
    cat = jnp.concatenate(xs, axis=1) if len(xs) > 1 else xs[0]
    c16 = _rup(c, 16)
    cap = _cap(n_out, k)
    n_chunks = -(-c16 // cap)
    ccw = _rup(-(-c16 // n_chunks), 16)
    cpad = n_chunks * ccw
    xp = jnp.pad(cat, ((0, 0), (0, cpad - c)))
    if n_chunks > 1:
        xt = xp.reshape(n, n_chunks, ccw).transpose(1, 0, 2)
    else:
        xt = xp.reshape(1, n, ccw)
    part = _na_fn(n, n_out, n_chunks, ccw, k)(xt, srcr, dstr)
    return (part, n_chunks, ccw, tuple(offs))


def _sub_groups(nam, i, W):
    part, n_chunks, ccw, offs = nam
    o, hi_s = offs[i], offs[i + 1]
    groups = []
    for cc in range(n_chunks):
        lo, hi = cc * ccw, (cc + 1) * ccw
        a, bnd = max(lo, o), min(hi, hi_s)
        if a >= bnd:
            continue
        wrows = W[a - o:bnd - o]
        wpad = jnp.pad(wrows, ((a - lo, ccw - (bnd - lo)), (0, 0)))
        groups.append(([part[cc, 0], part[cc, 1]], wpad))
    return groups


@functools.lru_cache(maxsize=None)
def _gather_fn(n_tab, c, k):
    mesh = plsc.VectorSubcoreMesh(core_axis_name="c", subcore_axis_name="s")
    m_pad = _NW * k * _EB

    def body(x_h, idxr_h, out_h, idx_v, rows_v, sem):
        ci = lax.axis_index("c")
        si = lax.axis_index("s")
        wid = ci * _NS + si
        pltpu.sync_copy(idxr_h.at[wid], idx_v)
        base = wid * (k * _EB)

        def _e(j, c2):
            pltpu.async_copy(x_h.at[idx_v.at[j]], rows_v, sem).wait()
            pltpu.sync_copy(rows_v, out_h.at[pl.ds(base + j * _EB, _EB)])
            return c2
        lax.fori_loop(0, k, _e, 0)

    return pl.kernel(
        body,
        out_type=jax.ShapeDtypeStruct((m_pad, c), _F32),
        mesh=mesh,
        compiler_params=pltpu.CompilerParams(use_tc_tiling_on_sc=False),
        scratch_types=[
            pltpu.VMEM((k, _EB), jnp.int32),
            pltpu.VMEM((_EB, c), _F32),
            pltpu.SemaphoreType.DMA,
        ],
    )


_BN = 512


def _dense(groups, b=None, res=(), inner_relu=False, outer_relu=False):
    n = groups[0][0][0].shape[0]
    co = groups[0][1].shape[1]
    nb = -(-n // _BN)
    xs_flat, ws, xcounts = [], [], []
    for xs, W in groups:
        xs_flat += list(xs)
        ws.append(W)
        xcounts.append(len(xs))
    res = list(res)
    ops = xs_flat + ws + res + ([b.reshape(1, co)] if b is not None else [])
    in_specs = (
        [pl.BlockSpec((_BN, x.shape[1]), lambda i: (i, 0)) for x in xs_flat]
        + [pl.BlockSpec(W.shape, lambda i: (0, 0)) for W in ws]
        + [pl.BlockSpec((_BN, co), lambda i: (i, 0)) for _ in res]
        + ([pl.BlockSpec((1, co), lambda i: (0, 0))] if b is not None else [])
    )
    nxs, nws, nres = len(xs_flat), len(ws), len(res)

    def body(*refs):
        out_ref = refs[-1]
        rs = refs[:-1]
        xi = 0
        acc = None
        for gidx, cnt in enumerate(xcounts):
            xsum = rs[xi][...]
            for t in range(1, cnt):
                xsum = xsum + rs[xi + t][...]
            xi += cnt
            d = jnp.dot(xsum, rs[nxs + gidx][...],
                        preferred_element_type=_F32)
            acc = d if acc is None else acc + d
        if b is not None:
            acc = acc + rs[nxs + nws + nres][...]
        if inner_relu:
            acc = jnp.maximum(acc, 0.0)
        for t in range(nres):
            acc = acc + rs[nxs + nws + t][...]
        if outer_relu:
            acc = jnp.maximum(acc, 0.0)
        out_ref[...] = acc

    return pl.pallas_call(
        body,
        grid=(nb,),
        in_specs=in_specs,
        out_specs=pl.BlockSpec((_BN, co), lambda i: (i, 0)),
        out_shape=jax.ShapeDtypeStruct((n, co), _F32),
    )(*ops)


def _mconv_g(xs, ps, g, relu=False, reses=None, outer_relu=False):
    nam = _na_multi(list(xs), g)
    outs = []
    for i, (x, p) in enumerate(zip(xs, ps)):
        groups = [([x], p['Ws'])] + _sub_groups(nam, i, p['Wn'])
        r = [] if reses is None else [reses[i]]
        outs.append(_dense(groups, b=p['b'], res=r, inner_relu=relu,
                           outer_relu=outer_relu))
    return outs


def _irb_g(xs, ps, g, outer_relu):
    outs = [_dense([([x], p['c00']['W'])], b=p['c00']['b'], inner_relu=True)
            for x, p in zip(xs, ps)]
    nam = _na_multi(outs + list(xs), g)
    nb = len(xs)
    m1s, ts = [], []
    for i, (x, p) in enumerate(zip(xs, ps)):
        g01 = [([outs[i]], p['c01']['Ws'])] + _sub_groups(nam, i,
                                                          p['c01']['Wn'])
        m1s.append(_dense(g01, b=p['c01']['b'], inner_relu=True))
        g10 = [([x], p['c10']['Ws'])] + _sub_groups(nam, nb + i,
                                                    p['c10']['Wn'])
        ts.append(_dense(g10, b=p['c10']['b'], inner_relu=True))
    nam2 = _na_multi(ts, g)
    res = []
    for i, (x, p) in enumerate(zip(xs, ps)):
        c2 = x.shape[1] // 2
        out0 = _dense([([m1s[i]], p['c02']['W'])], b=p['c02']['b'],
                      inner_relu=True, res=[x[:, :c2]],
                      outer_relu=outer_relu)
        g11 = [([ts[i]], p['c11']['Ws'])] + _sub_groups(nam2, i,
                                                        p['c11']['Wn'])
        out1 = _dense(g11, b=p['c11']['b'], inner_relu=True,
                      res=[x[:, c2:]], outer_relu=outer_relu)
        res.append(jnp.concatenate([out0, out1], axis=1))
    return res


def kernel(attr_feat, coords_feat, edge_index, up_src, edge_index_up,
           cross_src, cross_dst, edge_index_pruned, params, gt_n):
    p = params
    n = coords_feat.shape[0]
    m = up_src.shape[0]
    gt_ns = 40000

    gN = _graph(edge_index[0], edge_index[1], n)
    gU = _graph(edge_index_up[0], edge_index_up[1], m)
    gC = _graph(cross_src, cross_dst, gt_ns)
    gP = _graph(edge_index_pruned[0], edge_index_pruned[1], gt_ns)

    c, a = coords_feat, attr_feat
    for li in range(3):
        cv = _mconv_g([c, a], [p['coords_conv%d' % li],
                               p['attr_conv%d' % li]], gN)
        c, a = _irb_g(cv, [p['coords_res%d' % li],
                           p['attr_res%d' % li]], gN, outer_relu=True)
    c, a = _mconv_g([c, a], [p['coords_conv3'], p['attr_conv3']], gN)

    f = jnp.concatenate([c, a], axis=1)
    f = _mconv_g([f], [p['fusion0']], gN, relu=True)[0]
    f = _mconv_g([f], [p['fusion1']], gN)[0]
    h = f.shape[1] // 2
    cpart, apart = f[:, :h], f[:, h:]

    cw = _dense([([cpart], p['coords_up']['W'])], b=p['coords_up']['b'])
    m_pad = _rup(m, _NW * _EB)
    k_up = m_pad // (_NW * _EB)
    idxr = jnp.pad(up_src, (0, m_pad - m)).reshape(_NW, k_up, _EB)
    up = _gather_fn(n, cw.shape[1], k_up)(cw, idxr)[:m]

    cn = _mconv_g([up], [p['coords_convout']], gU, relu=True)[0]
    cn = _irb_g([cn], [p['coords_res3']], gU, outer_relu=False)[0]
    cls = _mconv_g([cn], [p['coords_cls']], gU)[0]

    nam = _na_multi([apart], gC)
    t = _dense(_sub_groups(nam, 0, p['attr_target']['Wn']),
               b=p['attr_target']['b'])
    out = _mconv_g([t], [p['attr_up_convout']], gP, relu=True)[0]
    out = _mconv_g([out], [p['conv_out']], gP)[0]
    out = out[:gt_ns] + (jnp.asarray(gt_n) - gt_ns).astype(out.dtype)
    return (out, cls)


# speedup vs baseline: 1.3644x; 1.3644x over previous
import functools

import jax
import jax.numpy as jnp
from jax import lax
from jax.experimental import pallas as pl
from jax.experimental.pallas import tpu as pltpu
from jax.experimental.pallas import tpu_sc as plsc

_NC, _NS = 2, 16
_NW = _NC * _NS
_EB = 128
_ZR = 64
_F32 = jnp.float32
_SPW = 2_060_000


def _rup(v, m):
    return (v + m - 1) // m * m


def _cap(n_out, k):
    n_acc = _rup(n_out, 128) + 128
    fixed = 16 * 2 * 8 * _EB
    per_ccw = n_acc + 16 * (8 * _EB + _ZR)
    return max(16, (_SPW - fixed) // per_ccw // 16 * 16)


@functools.lru_cache(maxsize=None)
def _na_fn(n_src, n_out, n_chunks, ccw, k):
    n_op = _rup(n_out, 128)
    n_acc = n_op + 128
    stripe = n_acc // _NS
    rows_out = n_op // _NS
    nfull, rem = divmod(stripe, _ZR)
    mesh = plsc.VectorSubcoreMesh(core_axis_name="c", subcore_axis_name="s")

    nw = k // 4

    def body(x_h, srcr_h, dstr_h, out_h, acc_sh, idxs_b, idxd_b, rows_b,
             zb_v, si0, si1, *sg):
        ci = lax.axis_index("c")
        si = lax.axis_index("s")
        wid = ci * _NS + si
        myis = srcr_h.at[wid]
        myid = dstr_h.at[wid]

        def _z(r, carry):
            for t in range(ccw // 16):
                zb_v[r, pl.ds(t * 16, 16)] = jnp.zeros((16,), _F32)
            return carry
        lax.fori_loop(0, _ZR, _z, 0)

        zbase = si * stripe
        obase = si * rows_out

        def ifetch(w, q, sem):
            pltpu.async_copy(myis.at[pl.ds(4 * w, 4)],
                             idxs_b.at[pl.ds(4 * q, 4)], sem)
            pltpu.async_copy(myid.at[pl.ds(4 * w, 4)],
                             idxd_b.at[pl.ds(4 * q, 4)], sem)

        def iwait(q, sem):
            pltpu.make_async_copy(myis.at[pl.ds(0, 4)],
                                  idxs_b.at[pl.ds(4 * q, 4)], sem).wait()
            pltpu.make_async_copy(myid.at[pl.ds(0, 4)],
                                  idxd_b.at[pl.ds(4 * q, 4)], sem).wait()

        def _chunk(cc, carry):
            def _zc(t, c2):
                pltpu.sync_copy(zb_v, acc_sh.at[pl.ds(zbase + t * _ZR, _ZR)])
                return c2
            lax.fori_loop(0, nfull, _zc, 0)
            if rem:
                pltpu.sync_copy(zb_v.at[pl.ds(0, rem)],
                                acc_sh.at[pl.ds(zbase + nfull * _ZR, rem)])
            plsc.subcore_barrier()

            xcc = x_h.at[cc]

            def gissue(q, r):
                pltpu.async_copy(xcc.at[idxs_b.at[4 * q + r]],
                                 rows_b.at[4 * q + r], sg[4 * q + r])

            def gdrain(q, r):
                pltpu.make_async_copy(xcc.at[pl.ds(0, _EB)],
                                      rows_b.at[4 * q + r],
                                      sg[4 * q + r]).wait()

            def scat(q, r):
                pltpu.sync_copy(rows_b.at[4 * q + r],
                                acc_sh.at[idxd_b.at[4 * q + r]], add=True)

            ifetch(0, 0, si0)
            ifetch(1, 1, si1)
            iwait(0, si0)
            for r in range(4):
                gissue(0, r)

            def _ww(ww, carry2):
                w0 = 2 * ww
                iwait(1, si1)
                for r in range(4):
                    gissue(1, r)
                for r in range(4):
                    gdrain(0, r)
                    scat(0, r)

                @pl.when(w0 + 2 < nw)
                def _():
                    ifetch(w0 + 2, 0, si0)

                w1 = w0 + 1

                @pl.when(w1 + 1 < nw)
                def _():
                    iwait(0, si0)
                    for r in range(4):
                        gissue(0, r)
                for r in range(4):
                    gdrain(1, r)
                    scat(1, r)

                @pl.when(w1 + 2 < nw)
                def _():
                    ifetch(w1 + 2, 1, si1)
                return carry2
            lax.fori_loop(0, nw // 2, _ww, 0)
            plsc.subcore_barrier()

            pltpu.sync_copy(acc_sh.at[pl.ds(obase, rows_out)],
                            out_h.at[cc, ci, pl.ds(obase, rows_out)])
            plsc.subcore_barrier()
            return carry
        lax.fori_loop(0, n_chunks, _chunk, 0)

    return pl.kernel(
        body,
        out_type=jax.ShapeDtypeStruct((n_chunks, _NC, n_op, ccw), _F32),
        mesh=mesh,
        compiler_params=pltpu.CompilerParams(use_tc_tiling_on_sc=False),
        scratch_types=[
            pltpu.VMEM_SHARED((n_acc, ccw), _F32),
            pltpu.VMEM((8, _EB), jnp.int32),
            pltpu.VMEM((8, _EB), jnp.int32),
            pltpu.VMEM((8, _EB, ccw), _F32),
            pltpu.VMEM((_ZR, ccw), _F32),
        ] + [pltpu.SemaphoreType.DMA] * 10,
    )


def _graph(src, dst, n_out):
    e = src.shape[0]
    epad = _rup(e, _NW * _EB * 8)
    k = epad // (_NW * _EB)
    srcr = jnp.pad(src, (0, epad - e)).reshape(_NW, k, _EB)
    dstr = jnp.pad(dst, (0, epad - e),
                   constant_values=n_out).reshape(_NW, k, _EB)
    return (srcr, dstr, k, n_out)


def _na_multi(xs, g):
    srcr, dstr, k, n_out = g
    n = xs[0].shape[0]
    widths = [x.shape[1] for x in xs]
    offs = [0]
    for w in widths:
        offs.append(offs[-1] + w)
    c = offs[-1]
    cat = jnp.concatenate(xs, axis=1) if len(xs) > 1 else xs[0]
    c16 = _rup(c, 16)
    cap = _cap(n_out, k)
    n_chunks = -(-c16 // cap)
    ccw = _rup(-(-c16 // n_chunks), 16)
    cpad = n_chunks * ccw
    xp = jnp.pad(cat, ((0, 0), (0, cpad - c)))
    if n_chunks > 1:
        xt = xp.reshape(n, n_chunks, ccw).transpose(1, 0, 2)
    else:
        xt = xp.reshape(1, n, ccw)
    part = _na_fn(n, n_out, n_chunks, ccw, k)(xt, srcr, dstr)
    return (part, n_chunks, ccw, tuple(offs))


def _sub_groups(nam, i, W):
    part, n_chunks, ccw, offs = nam
    o, hi_s = offs[i], offs[i + 1]
    groups = []
    for cc in range(n_chunks):
        lo, hi = cc * ccw, (cc + 1) * ccw
        a, bnd = max(lo, o), min(hi, hi_s)
        if a >= bnd:
            continue
        wrows = W[a - o:bnd - o]
        wpad = jnp.pad(wrows, ((a - lo, ccw - (bnd - lo)), (0, 0)))
        groups.append(([part[cc, 0], part[cc, 1]], wpad))
    return groups


@functools.lru_cache(maxsize=None)
def _gather_fn(n_tab, c, k):
    mesh = plsc.VectorSubcoreMesh(core_axis_name="c", subcore_axis_name="s")
    m_pad = _NW * k * _EB

    def body(x_h, idxr_h, out_h, idx_v, rows_v, sem):
        ci = lax.axis_index("c")
        si = lax.axis_index("s")
        wid = ci * _NS + si
        pltpu.sync_copy(idxr_h.at[wid], idx_v)
        base = wid * (k * _EB)

        def _e(j, c2):
            pltpu.async_copy(x_h.at[idx_v.at[j]], rows_v, sem).wait()
            pltpu.sync_copy(rows_v, out_h.at[pl.ds(base + j * _EB, _EB)])
            return c2
        lax.fori_loop(0, k, _e, 0)

    return pl.kernel(
        body,
        out_type=jax.ShapeDtypeStruct((m_pad, c), _F32),
        mesh=mesh,
        compiler_params=pltpu.CompilerParams(use_tc_tiling_on_sc=False),
        scratch_types=[
            pltpu.VMEM((k, _EB), jnp.int32),
            pltpu.VMEM((_EB, c), _F32),
            pltpu.SemaphoreType.DMA,
        ],
    )


_BN = 512


def _dense(groups, b=None, res=(), inner_relu=False, outer_relu=False):
    n = groups[0][0][0].shape[0]
    co = groups[0][1].shape[1]
    nb = -(-n // _BN)
    xs_flat, ws, xcounts = [], [], []
    for xs, W in groups:
        xs_flat += list(xs)
        ws.append(W)
        xcounts.append(len(xs))
    res = list(res)
    ops = xs_flat + ws + res + ([b.reshape(1, co)] if b is not None else [])
    in_specs = (
        [pl.BlockSpec((_BN, x.shape[1]), lambda i: (i, 0)) for x in xs_flat]
        + [pl.BlockSpec(W.shape, lambda i: (0, 0)) for W in ws]
        + [pl.BlockSpec((_BN, co), lambda i: (i, 0)) for _ in res]
        + ([pl.BlockSpec((1, co), lambda i: (0, 0))] if b is not None else [])
    )
    nxs, nws, nres = len(xs_flat), len(ws), len(res)

    def body(*refs):
        out_ref = refs[-1]
        rs = refs[:-1]
        xi = 0
        acc = None
        for gidx, cnt in enumerate(xcounts):
            xsum = rs[xi][...]
            for t in range(1, cnt):
                xsum = xsum + rs[xi + t][...]
            xi += cnt
            d = jnp.dot(xsum, rs[nxs + gidx][...],
                        preferred_element_type=_F32)
            acc = d if acc is None else acc + d
        if b is not None:
            acc = acc + rs[nxs + nws + nres][...]
        if inner_relu:
            acc = jnp.maximum(acc, 0.0)
        for t in range(nres):
            acc = acc + rs[nxs + nws + t][...]
        if outer_relu:
            acc = jnp.maximum(acc, 0.0)
        out_ref[...] = acc

    return pl.pallas_call(
        body,
        grid=(nb,),
        in_specs=in_specs,
        out_specs=pl.BlockSpec((_BN, co), lambda i: (i, 0)),
        out_shape=jax.ShapeDtypeStruct((n, co), _F32),
    )(*ops)


def _mconv_g(xs, ps, g, relu=False, reses=None, outer_relu=False,
             pre=False):
    if pre:
        nas = [_dense([([x], p['Wn'])]) for x, p in zip(xs, ps)]
    else:
        nas = list(xs)
    nam = _na_multi(nas, g)
    outs = []
    for i, (x, p) in enumerate(zip(xs, ps)):
        wn = jnp.eye(p['Wn'].shape[1], dtype=_F32) if pre else p['Wn']
        groups = [([x], p['Ws'])] + _sub_groups(nam, i, wn)
        r = [] if reses is None else [reses[i]]
        outs.append(_dense(groups, b=p['b'], res=r, inner_relu=relu,
                           outer_relu=outer_relu))
    return outs


def _irb_g(xs, ps, g, outer_relu):
    outs = [_dense([([x], p['c00']['W'])], b=p['c00']['b'], inner_relu=True)
            for x, p in zip(xs, ps)]
    xws = [_dense([([x], p['c10']['Wn'])]) for x, p in zip(xs, ps)]
    nam = _na_multi(outs + xws, g)
    nb = len(xs)
    m1s, ts = [], []
    for i, (x, p) in enumerate(zip(xs, ps)):
        g01 = [([outs[i]], p['c01']['Ws'])] + _sub_groups(nam, i,
                                                          p['c01']['Wn'])
        m1s.append(_dense(g01, b=p['c01']['b'], inner_relu=True))
        eye10 = jnp.eye(p['c10']['Wn'].shape[1], dtype=_F32)
        g10 = [([x], p['c10']['Ws'])] + _sub_groups(nam, nb + i, eye10)
        ts.append(_dense(g10, b=p['c10']['b'], inner_relu=True))
    nam2 = _na_multi(ts, g)
    res = []
    for i, (x, p) in enumerate(zip(xs, ps)):
        c2 = x.shape[1] // 2
        out0 = _dense([([m1s[i]], p['c02']['W'])], b=p['c02']['b'],
                      inner_relu=True, res=[x[:, :c2]],
                      outer_relu=outer_relu)
        g11 = [([ts[i]], p['c11']['Ws'])] + _sub_groups(nam2, i,
                                                        p['c11']['Wn'])
        out1 = _dense(g11, b=p['c11']['b'], inner_relu=True,
                      res=[x[:, c2:]], outer_relu=outer_relu)
        res.append(jnp.concatenate([out0, out1], axis=1))
    return res


def kernel(attr_feat, coords_feat, edge_index, up_src, edge_index_up,
           cross_src, cross_dst, edge_index_pruned, params, gt_n):
    p = params
    n = coords_feat.shape[0]
    m = up_src.shape[0]
    gt_ns = 40000

    gN = _graph(edge_index[0], edge_index[1], n)
    gU = _graph(edge_index_up[0], edge_index_up[1], m)
    gC = _graph(cross_src, cross_dst, gt_ns)
    gP = _graph(edge_index_pruned[0], edge_index_pruned[1], gt_ns)

    c, a = coords_feat, attr_feat
    for li in range(3):
        cv = _mconv_g([c, a], [p['coords_conv%d' % li],
                               p['attr_conv%d' % li]], gN)
        c, a = _irb_g(cv, [p['coords_res%d' % li],
                           p['attr_res%d' % li]], gN, outer_relu=True)
    c, a = _mconv_g([c, a], [p['coords_conv3'], p['attr_conv3']], gN)

    f = jnp.concatenate([c, a], axis=1)
    f = _mconv_g([f], [p['fusion0']], gN, relu=True, pre=True)[0]
    f = _mconv_g([f], [p['fusion1']], gN)[0]
    h = f.shape[1] // 2
    cpart, apart = f[:, :h], f[:, h:]

    cw = _dense([([cpart], p['coords_up']['W'])], b=p['coords_up']['b'])
    m_pad = _rup(m, _NW * _EB)
    k_up = m_pad // (_NW * _EB)
    idxr = jnp.pad(up_src, (0, m_pad - m)).reshape(_NW, k_up, _EB)
    up = _gather_fn(n, cw.shape[1], k_up)(cw, idxr)[:m]

    cn = _mconv_g([up], [p['coords_convout']], gU, relu=True, pre=True)[0]
    cn = _irb_g([cn], [p['coords_res3']], gU, outer_relu=False)[0]
    cls = _mconv_g([cn], [p['coords_cls']], gU, pre=True)[0]

    nam = _na_multi([apart], gC)
    t = _dense(_sub_groups(nam, 0, p['attr_target']['Wn']),
               b=p['attr_target']['b'])
    out = _mconv_g([t], [p['attr_up_convout']], gP, relu=True, pre=True)[0]
    out = _mconv_g([out], [p['conv_out']], gP, pre=True)[0]
    out = out[:gt_ns] + (jnp.asarray(gt_n) - gt_ns).astype(out.dtype)
    return (out, cls)


# speedup vs baseline: 1.5202x; 1.1142x over previous
import functools

import jax
import jax.numpy as jnp
from jax import lax
from jax.experimental import pallas as pl
from jax.experimental.pallas import tpu as pltpu
from jax.experimental.pallas import tpu_sc as plsc

_NC, _NS = 2, 16
_NW = _NC * _NS
_EB = 128
_ZR = 64
_F32 = jnp.float32
_SPW = 2_060_000


def _rup(v, m):
    return (v + m - 1) // m * m


def _cap(n_out, k):
    n_acc = _rup(n_out, 128) + 128
    fixed = 16 * 2 * 8 * _EB
    per_ccw = n_acc + 16 * (8 * _EB + _ZR)
    return max(16, (_SPW - fixed) // per_ccw // 16 * 16)


@functools.lru_cache(maxsize=None)
def _na_fn(n_src, n_out, n_chunks, ccw, k):
    n_op = _rup(n_out, 128)
    n_acc = n_op + 128
    stripe = n_acc // _NS
    rows_out = n_op // _NS
    nfull, rem = divmod(stripe, _ZR)
    mesh = plsc.VectorSubcoreMesh(core_axis_name="c", subcore_axis_name="s")

    nw = k // 4

    def body(x_h, srcr_h, dstr_h, out_h, acc_sh, idxs_b, idxd_b, rows_b,
             zb_v, si0, si1, *sg):
        ci = lax.axis_index("c")
        si = lax.axis_index("s")
        wid = ci * _NS + si
        myis = srcr_h.at[wid]
        myid = dstr_h.at[wid]

        def _z(r, carry):
            for t in range(ccw // 16):
                zb_v[r, pl.ds(t * 16, 16)] = jnp.zeros((16,), _F32)
            return carry
        lax.fori_loop(0, _ZR, _z, 0)

        zbase = si * stripe
        obase = si * rows_out

        def ifetch(w, q, sem):
            pltpu.async_copy(myis.at[pl.ds(4 * w, 4)],
                             idxs_b.at[pl.ds(4 * q, 4)], sem)
            pltpu.async_copy(myid.at[pl.ds(4 * w, 4)],
                             idxd_b.at[pl.ds(4 * q, 4)], sem)

        def iwait(q, sem):
            pltpu.make_async_copy(myis.at[pl.ds(0, 4)],
                                  idxs_b.at[pl.ds(4 * q, 4)], sem).wait()
            pltpu.make_async_copy(myid.at[pl.ds(0, 4)],
                                  idxd_b.at[pl.ds(4 * q, 4)], sem).wait()

        def _chunk(cc, carry):
            def _zc(t, c2):
                pltpu.sync_copy(zb_v, acc_sh.at[pl.ds(zbase + t * _ZR, _ZR)])
                return c2
            lax.fori_loop(0, nfull, _zc, 0)
            if rem:
                pltpu.sync_copy(zb_v.at[pl.ds(0, rem)],
                                acc_sh.at[pl.ds(zbase + nfull * _ZR, rem)])
            plsc.subcore_barrier()

            xcc = x_h.at[cc]

            def gissue(q, r):
                pltpu.async_copy(xcc.at[idxs_b.at[4 * q + r]],
                                 rows_b.at[4 * q + r], sg[4 * q + r])

            def gdrain(q, r):
                pltpu.make_async_copy(xcc.at[pl.ds(0, _EB)],
                                      rows_b.at[4 * q + r],
                                      sg[4 * q + r]).wait()

            def scat(q, r):
                pltpu.sync_copy(rows_b.at[4 * q + r],
                                acc_sh.at[idxd_b.at[4 * q + r]], add=True)

            ifetch(0, 0, si0)
            ifetch(1, 1, si1)
            iwait(0, si0)
            for r in range(4):
                gissue(0, r)

            def _ww(ww, carry2):
                w0 = 2 * ww
                iwait(1, si1)
                for r in range(4):
                    gissue(1, r)
                for r in range(4):
                    gdrain(0, r)
                    scat(0, r)

                @pl.when(w0 + 2 < nw)
                def _():
                    ifetch(w0 + 2, 0, si0)

                w1 = w0 + 1

                @pl.when(w1 + 1 < nw)
                def _():
                    iwait(0, si0)
                    for r in range(4):
                        gissue(0, r)
                for r in range(4):
                    gdrain(1, r)
                    scat(1, r)

                @pl.when(w1 + 2 < nw)
                def _():
                    ifetch(w1 + 2, 1, si1)
                return carry2
            lax.fori_loop(0, nw // 2, _ww, 0)
            plsc.subcore_barrier()

            pltpu.sync_copy(acc_sh.at[pl.ds(obase, rows_out)],
                            out_h.at[cc, ci, pl.ds(obase, rows_out)])
            plsc.subcore_barrier()
            return carry
        lax.fori_loop(0, n_chunks, _chunk, 0)

    return pl.kernel(
        body,
        out_type=jax.ShapeDtypeStruct((n_chunks, _NC, n_op, ccw), _F32),
        mesh=mesh,
        compiler_params=pltpu.CompilerParams(use_tc_tiling_on_sc=False),
        scratch_types=[
            pltpu.VMEM_SHARED((n_acc, ccw), _F32),
            pltpu.VMEM((8, _EB), jnp.int32),
            pltpu.VMEM((8, _EB), jnp.int32),
            pltpu.VMEM((8, _EB, ccw), _F32),
            pltpu.VMEM((_ZR, ccw), _F32),
        ] + [pltpu.SemaphoreType.DMA] * 10,
    )


def _graph(src, dst, n_out):
    e = src.shape[0]
    epad = _rup(e, _NW * _EB * 8)
    k = epad // (_NW * _EB)
    srcr = jnp.pad(src, (0, epad - e)).reshape(_NW, k, _EB)
    dstr = jnp.pad(dst, (0, epad - e),
                   constant_values=n_out).reshape(_NW, k, _EB)
    return (srcr, dstr, k, n_out)


def _na_multi(xs, g):
    srcr, dstr, k, n_out = g
    n = xs[0].shape[0]
    widths = [x.shape[1] for x in xs]
    offs = [0]
    for w in widths:
        offs.append(offs[-1] + w)
    c = offs[-1]
    cat = jnp.concatenate(xs, axis=1) if len(xs) > 1 else xs[0]
    c16 = _rup(c, 16)
    cap = _cap(n_out, k)
    n_chunks = -(-c16 // cap)
    ccw = _rup(-(-c16 // n_chunks), 16)
    cpad = n_chunks * ccw
    xp = jnp.pad(cat, ((0, 0), (0, cpad - c)))
    if n_chunks > 1:
        xt = xp.reshape(n, n_chunks, ccw).transpose(1, 0, 2)
    else:
        xt = xp.reshape(1, n, ccw)
    part = _na_fn(n, n_out, n_chunks, ccw, k)(xt, srcr, dstr)
    return (part, n_chunks, ccw, tuple(offs))


def _sub_range(nam, o, w, W):
    part, n_chunks, ccw, _ = nam
    hi_s = o + w
    groups = []
    for cc in range(n_chunks):
        lo, hi = cc * ccw, (cc + 1) * ccw
        a, bnd = max(lo, o), min(hi, hi_s)
        if a >= bnd:
            continue
        wrows = W[a - o:bnd - o]
        wpad = jnp.pad(wrows, ((a - lo, ccw - (bnd - lo)), (0, 0)))
        groups.append(([part[cc, 0], part[cc, 1]], wpad))
    return groups


@functools.lru_cache(maxsize=None)
def _gather_fn(n_tab, c, k):
    mesh = plsc.VectorSubcoreMesh(core_axis_name="c", subcore_axis_name="s")
    m_pad = _NW * k * _EB

    def body(x_h, idxr_h, out_h, idx_v, rows_v, sem):
        ci = lax.axis_index("c")
        si = lax.axis_index("s")
        wid = ci * _NS + si
        pltpu.sync_copy(idxr_h.at[wid], idx_v)
        base = wid * (k * _EB)

        def _e(j, c2):
            pltpu.async_copy(x_h.at[idx_v.at[j]], rows_v, sem).wait()
            pltpu.sync_copy(rows_v, out_h.at[pl.ds(base + j * _EB, _EB)])
            return c2
        lax.fori_loop(0, k, _e, 0)

    return pl.kernel(
        body,
        out_type=jax.ShapeDtypeStruct((m_pad, c), _F32),
        mesh=mesh,
        compiler_params=pltpu.CompilerParams(use_tc_tiling_on_sc=False),
        scratch_types=[
            pltpu.VMEM((k, _EB), jnp.int32),
            pltpu.VMEM((_EB, c), _F32),
            pltpu.SemaphoreType.DMA,
        ],
    )


_BN = 512


def _xb(x, jb=0, w=None):
    return (x, jb, x.shape[1] if w is None else w)


def _seg(groups, b=None, res=(), inner_relu=False, outer_relu=False):
    groups = [([e if isinstance(e, tuple) else _xb(e) for e in xs], W)
              for xs, W in groups]
    res = [e if isinstance(e, tuple) else _xb(e) for e in res]
    return dict(groups=groups, b=b, res=res, ir=inner_relu, orr=outer_relu)


def _dense_multi(segments):
    n = segments[0]['groups'][0][0][0][0].shape[0]
    widths = [s['groups'][0][1].shape[1] for s in segments]
    co = sum(widths)
    nb = -(-n // _BN)
    ops, in_specs, plan = [], [], []

    def add_op(arr, spec):
        ops.append(arr)
        in_specs.append(spec)
        return len(ops) - 1

    def add_x(x, jb, w):
        if w == x.shape[1]:
            return add_op(x, pl.BlockSpec((_BN, w), lambda i: (i, 0)))
        if w % 128 == 0:
            return add_op(x, pl.BlockSpec((_BN, w), lambda i, j=jb: (i, j)))
        xsl = lax.slice_in_dim(x, jb * w, (jb + 1) * w, axis=1)
        return add_op(xsl, pl.BlockSpec((_BN, w), lambda i: (i, 0)))

    for s in segments:
        gplan = []
        for xs, W in s['groups']:
            xids = [add_x(x, jb, w) for x, jb, w in xs]
            wid = add_op(W, pl.BlockSpec(W.shape, lambda i: (0, 0)))
            gplan.append((xids, wid))
        rids = [add_x(x, jb, w) for x, jb, w in s['res']]
        bid = None
        if s['b'] is not None:
            wseg = s['groups'][0][1].shape[1]
            bid = add_op(s['b'].reshape(1, wseg),
                         pl.BlockSpec((1, wseg), lambda i: (0, 0)))
        plan.append((gplan, rids, bid, s['ir'], s['orr']))

    def body(*refs):
        out_ref = refs[-1]
        rs = refs[:-1]
        off = 0
        for (gplan, rids, bid, ir, orr), w in zip(plan, widths):
            acc = None
            for xids, wid in gplan:
                xsum = rs[xids[0]][...]
                for t in xids[1:]:
                    xsum = xsum + rs[t][...]
                d = jnp.dot(xsum, rs[wid][...], preferred_element_type=_F32)
                acc = d if acc is None else acc + d
            if bid is not None:
                acc = acc + rs[bid][...]
            if ir:
                acc = jnp.maximum(acc, 0.0)
            for t in rids:
                acc = acc + rs[t][...]
            if orr:
                acc = jnp.maximum(acc, 0.0)
            out_ref[:, off:off + w] = acc
            off += w

    return pl.pallas_call(
        body,
        grid=(nb,),
        in_specs=in_specs,
        out_specs=pl.BlockSpec((_BN, co), lambda i: (i, 0)),
        out_shape=jax.ShapeDtypeStruct((n, co), _F32),
    )(*ops)


def _dense(groups, b=None, res=(), inner_relu=False, outer_relu=False):
    return _dense_multi([_seg(groups, b, res, inner_relu, outer_relu)])


def _gconv(xents, ps, g, na_list, na_offs, na_ws, subWs, relu=False):
    nam = _na_multi(na_list, g)
    segs = []
    for i, p in enumerate(ps):
        groups = ([([xents[i]], p['Ws'])]
                  + _sub_range(nam, na_offs[i], na_ws[i], subWs[i]))
        segs.append(_seg(groups, b=p['b'], inner_relu=relu))
    return _dense_multi(segs)


def _eye(W):
    return jnp.eye(W.shape[1], dtype=_F32)


def _conv_blk(f, w_in, ps, g, relu=False, pre=False):
    nb = len(ps)
    xents = [_xb(f, i, w_in) for i in range(nb)]
    if pre:
        xn = _dense_multi([_seg([([xents[i]], ps[i]['Wn'])])
                           for i in range(nb)])
        ws = [p['Wn'].shape[1] for p in ps]
        offs = [sum(ws[:i]) for i in range(nb)]
        return _gconv(xents, ps, g, [xn], offs, ws,
                      [_eye(p['Wn']) for p in ps], relu=relu)
    return _gconv(xents, ps, g, [f], [i * w_in for i in range(nb)],
                  [w_in] * nb, [p['Wn'] for p in ps], relu=relu)


def _irb_blk(f, ps, g, outer_relu):
    nb = len(ps)
    c = f.shape[1] // nb
    c4, c2 = c // 4, c // 2
    A = _dense_multi(
        [_seg([([_xb(f, i, c)], p['c00']['W'])], b=p['c00']['b'],
              inner_relu=True) for i, p in enumerate(ps)]
        + [_seg([([_xb(f, i, c)], p['c10']['Wn'])])
           for i, p in enumerate(ps)])
    nam = _na_multi([A], g)
    B = _dense_multi(
        [_seg([([_xb(f, i, c)], p['c10']['Ws'])]
              + _sub_range(nam, (nb + i) * c4, c4, _eye(p['c10']['Wn'])),
              b=p['c10']['b'], inner_relu=True) for i, p in enumerate(ps)]
        + [_seg([([_xb(A, i, c4)], p['c01']['Ws'])]
                + _sub_range(nam, i * c4, c4, p['c01']['Wn']),
                b=p['c01']['b'], inner_relu=True)
           for i, p in enumerate(ps)])
    nam2 = _na_multi([B[:, :nb * c4]], g)
    segs = []
    for i, p in enumerate(ps):
        segs.append(_seg([([_xb(B, nb + i, c4)], p['c02']['W'])],
                         b=p['c02']['b'], inner_relu=True,
                         res=[_xb(f, 2 * i, c2)], outer_relu=outer_relu))
        segs.append(_seg([([_xb(B, i, c4)], p['c11']['Ws'])]
                         + _sub_range(nam2, i * c4, c4, p['c11']['Wn']),
                         b=p['c11']['b'], inner_relu=True,
                         res=[_xb(f, 2 * i + 1, c2)], outer_relu=outer_relu))
    return _dense_multi(segs)


def kernel(attr_feat, coords_feat, edge_index, up_src, edge_index_up,
           cross_src, cross_dst, edge_index_pruned, params, gt_n):
    p = params
    n = coords_feat.shape[0]
    m = up_src.shape[0]
    gt_ns = 40000

    gN = _graph(edge_index[0], edge_index[1], n)
    gU = _graph(edge_index_up[0], edge_index_up[1], m)
    gC = _graph(cross_src, cross_dst, gt_ns)
    gP = _graph(edge_index_pruned[0], edge_index_pruned[1], gt_ns)

    ps0 = [p['coords_conv0'], p['attr_conv0']]
    f = _gconv([_xb(coords_feat), _xb(attr_feat)], ps0, gN,
               [coords_feat, attr_feat], [0, 1], [1, 3],
               [q['Wn'] for q in ps0])
    f = _irb_blk(f, [p['coords_res0'], p['attr_res0']], gN, outer_relu=True)
    for li in (1, 2):
        f = _conv_blk(f, f.shape[1] // 2,
                      [p['coords_conv%d' % li], p['attr_conv%d' % li]], gN)
        f = _irb_blk(f, [p['coords_res%d' % li], p['attr_res%d' % li]], gN,
                     outer_relu=True)
    f = _conv_blk(f, f.shape[1] // 2,
                  [p['coords_conv3'], p['attr_conv3']], gN)

    f = _conv_blk(f, f.shape[1], [p['fusion0']], gN, relu=True, pre=True)
    f = _conv_blk(f, f.shape[1], [p['fusion1']], gN)
    h = f.shape[1] // 2

    cw = _dense([([_xb(f, 0, h)], p['coords_up']['W'])],
                b=p['coords_up']['b'])
    m_pad = _rup(m, _NW * _EB)
    k_up = m_pad // (_NW * _EB)
    idxr = jnp.pad(up_src, (0, m_pad - m)).reshape(_NW, k_up, _EB)
    up = _gather_fn(n, cw.shape[1], k_up)(cw, idxr)[:m]

    cn = _conv_blk(up, up.shape[1], [p['coords_convout']], gU, relu=True,
                   pre=True)
    cn = _irb_blk(cn, [p['coords_res3']], gU, outer_relu=False)
    cls = _conv_blk(cn, cn.shape[1], [p['coords_cls']], gU, pre=True)

    nam = _na_multi([f[:, h:]], gC)
    t = _dense(_sub_range(nam, 0, h, p['attr_target']['Wn']),
               b=p['attr_target']['b'])
    out = _conv_blk(t, t.shape[1], [p['attr_up_convout']], gP, relu=True,
                    pre=True)
    out = _conv_blk(out, out.shape[1], [p['conv_out']], gP, pre=True)
    out = out[:gt_ns] + (jnp.asarray(gt_n) - gt_ns).astype(out.dtype)
    return (out, cls)


# speedup vs baseline: 1.5245x; 1.0028x over previous
import functools

import jax
import jax.numpy as jnp
from jax import lax
from jax.experimental import pallas as pl
from jax.experimental.pallas import tpu as pltpu
from jax.experimental.pallas import tpu_sc as plsc

_NC, _NS = 2, 16
_NW = _NC * _NS
_EB = 128
_ZR = 64
_F32 = jnp.float32
_SPW = 2_060_000


def _rup(v, m):
    return (v + m - 1) // m * m


def _cap(n_out, k):
    n_acc = _rup(n_out, 128) + 128
    fixed = 16 * 2 * 8 * _EB
    per_ccw = n_acc + 16 * (8 * _EB + _ZR)
    return max(16, (_SPW - fixed) // per_ccw // 16 * 16)


@functools.lru_cache(maxsize=None)
def _na_fn(n_src, n_out, n_chunks, ccw, k):
    n_op = _rup(n_out, 128)
    n_acc = n_op + 128
    stripe = n_acc // _NS
    rows_out = n_op // _NS
    nfull, rem = divmod(stripe, _ZR)
    mesh = plsc.VectorSubcoreMesh(core_axis_name="c", subcore_axis_name="s")

    nw = k // 4

    def body(x_h, srcr_h, dstr_h, out_h, acc_sh, idxs_b, idxd_b, rows_b,
             zb_v, si0, si1, *sems):
        sg, ss = sems[:8], sems[8:]
        ci = lax.axis_index("c")
        si = lax.axis_index("s")
        wid = ci * _NS + si
        myis = srcr_h.at[wid]
        myid = dstr_h.at[wid]

        def _z(r, carry):
            for t in range(ccw // 16):
                zb_v[r, pl.ds(t * 16, 16)] = jnp.zeros((16,), _F32)
            return carry
        lax.fori_loop(0, _ZR, _z, 0)

        zbase = si * stripe
        obase = si * rows_out

        def ifetch(w, q, sem):
            pltpu.async_copy(myis.at[pl.ds(4 * w, 4)],
                             idxs_b.at[pl.ds(4 * q, 4)], sem)
            pltpu.async_copy(myid.at[pl.ds(4 * w, 4)],
                             idxd_b.at[pl.ds(4 * q, 4)], sem)

        def iwait(q, sem):
            pltpu.make_async_copy(myis.at[pl.ds(0, 4)],
                                  idxs_b.at[pl.ds(4 * q, 4)], sem).wait()
            pltpu.make_async_copy(myid.at[pl.ds(0, 4)],
                                  idxd_b.at[pl.ds(4 * q, 4)], sem).wait()

        def _chunk(cc, carry):
            def _zc(t, c2):
                pltpu.sync_copy(zb_v, acc_sh.at[pl.ds(zbase + t * _ZR, _ZR)])
                return c2
            lax.fori_loop(0, nfull, _zc, 0)
            if rem:
                pltpu.sync_copy(zb_v.at[pl.ds(0, rem)],
                                acc_sh.at[pl.ds(zbase + nfull * _ZR, rem)])
            plsc.subcore_barrier()

            xcc = x_h.at[cc]

            def gissue(q, r):
                pltpu.async_copy(xcc.at[idxs_b.at[4 * q + r]],
                                 rows_b.at[4 * q + r], sg[4 * q + r])

            def gdrain(q, r):
                pltpu.make_async_copy(xcc.at[pl.ds(0, _EB)],
                                      rows_b.at[4 * q + r],
                                      sg[4 * q + r]).wait()

            def scat(q, r):
                pltpu.async_copy(rows_b.at[4 * q + r],
                                 acc_sh.at[idxd_b.at[4 * q + r]],
                                 ss[4 * q + r])

            def sdrain(q, r):
                pltpu.make_async_copy(rows_b.at[4 * q + r],
                                      acc_sh.at[pl.ds(0, _EB)],
                                      ss[4 * q + r]).wait()

            ifetch(0, 0, si0)
            ifetch(1, 1, si1)
            iwait(0, si0)
            for r in range(4):
                gissue(0, r)

            def _ww(ww, carry2):
                w0 = 2 * ww
                iwait(1, si1)

                @pl.when(ww > 0)
                def _():
                    for r in range(4):
                        sdrain(1, r)
                for r in range(4):
                    gissue(1, r)
                for r in range(4):
                    gdrain(0, r)
                    scat(0, r)

                @pl.when(w0 + 2 < nw)
                def _():
                    ifetch(w0 + 2, 0, si0)

                w1 = w0 + 1

                @pl.when(w1 + 1 < nw)
                def _():
                    iwait(0, si0)
                    for r in range(4):
                        sdrain(0, r)
                    for r in range(4):
                        gissue(0, r)
                for r in range(4):
                    gdrain(1, r)
                    scat(1, r)

                @pl.when(w1 + 2 < nw)
                def _():
                    ifetch(w1 + 2, 1, si1)
                return carry2
            lax.fori_loop(0, nw // 2, _ww, 0)
            for r in range(4):
                sdrain(0, r)
            for r in range(4):
                sdrain(1, r)
            plsc.subcore_barrier()

            pltpu.sync_copy(acc_sh.at[pl.ds(obase, rows_out)],
                            out_h.at[cc, ci, pl.ds(obase, rows_out)])
            plsc.subcore_barrier()
            return carry
        lax.fori_loop(0, n_chunks, _chunk, 0)

    return pl.kernel(
        body,
        out_type=jax.ShapeDtypeStruct((n_chunks, _NC, n_op, ccw), _F32),
        mesh=mesh,
        compiler_params=pltpu.CompilerParams(use_tc_tiling_on_sc=False),
        scratch_types=[
            pltpu.VMEM_SHARED((n_acc, ccw), _F32),
            pltpu.VMEM((8, _EB), jnp.int32),
            pltpu.VMEM((8, _EB), jnp.int32),
            pltpu.VMEM((8, _EB, ccw), _F32),
            pltpu.VMEM((_ZR, ccw), _F32),
        ] + [pltpu.SemaphoreType.DMA] * 18,
    )


def _graph(src, dst, n_out):
    e = src.shape[0]
    epad = _rup(e, _NW * _EB * 8)
    k = epad // (_NW * _EB)
    srcr = jnp.pad(src, (0, epad - e)).reshape(_NW, k, _EB)
    dstr = jnp.pad(dst, (0, epad - e),
                   constant_values=n_out).reshape(_NW, k, _EB)
    return (srcr, dstr, k, n_out)


def _na_multi(xs, g):
    srcr, dstr, k, n_out = g
    n = xs[0].shape[0]
    widths = [x.shape[1] for x in xs]
    offs = [0]
    for w in widths:
        offs.append(offs[-1] + w)
    c = offs[-1]
    cat = jnp.concatenate(xs, axis=1) if len(xs) > 1 else xs[0]
    c16 = _rup(c, 16)
    cap = _cap(n_out, k)
    n_chunks = -(-c16 // cap)
    ccw = _rup(-(-c16 // n_chunks), 16)
    cpad = n_chunks * ccw
    xp = jnp.pad(cat, ((0, 0), (0, cpad - c)))
    if n_chunks > 1:
        xt = xp.reshape(n, n_chunks, ccw).transpose(1, 0, 2)
    else:
        xt = xp.reshape(1, n, ccw)
    part = _na_fn(n, n_out, n_chunks, ccw, k)(xt, srcr, dstr)
    return (part, n_chunks, ccw, tuple(offs))


def _sub_range(nam, o, w, W):
    part, n_chunks, ccw, _ = nam
    hi_s = o + w
    groups = []
    for cc in range(n_chunks):
        lo, hi = cc * ccw, (cc + 1) * ccw
        a, bnd = max(lo, o), min(hi, hi_s)
        if a >= bnd:
            continue
        wrows = W[a - o:bnd - o]
        wpad = jnp.pad(wrows, ((a - lo, ccw - (bnd - lo)), (0, 0)))
        groups.append(([part[cc, 0], part[cc, 1]], wpad))
    return groups


@functools.lru_cache(maxsize=None)
def _gather_fn(n_tab, c, k):
    mesh = plsc.VectorSubcoreMesh(core_axis_name="c", subcore_axis_name="s")
    m_pad = _NW * k * _EB

    def body(x_h, idxr_h, out_h, idx_v, rows_v, sem):
        ci = lax.axis_index("c")
        si = lax.axis_index("s")
        wid = ci * _NS + si
        pltpu.sync_copy(idxr_h.at[wid], idx_v)
        base = wid * (k * _EB)

        def _e(j, c2):
            pltpu.async_copy(x_h.at[idx_v.at[j]], rows_v, sem).wait()
            pltpu.sync_copy(rows_v, out_h.at[pl.ds(base + j * _EB, _EB)])
            return c2
        lax.fori_loop(0, k, _e, 0)

    return pl.kernel(
        body,
        out_type=jax.ShapeDtypeStruct((m_pad, c), _F32),
        mesh=mesh,
        compiler_params=pltpu.CompilerParams(use_tc_tiling_on_sc=False),
        scratch_types=[
            pltpu.VMEM((k, _EB), jnp.int32),
            pltpu.VMEM((_EB, c), _F32),
            pltpu.SemaphoreType.DMA,
        ],
    )


_BN = 512


def _xb(x, jb=0, w=None):
    return (x, jb, x.shape[1] if w is None else w)


def _seg(groups, b=None, res=(), inner_relu=False, outer_relu=False):
    groups = [([e if isinstance(e, tuple) else _xb(e) for e in xs], W)
              for xs, W in groups]
    res = [e if isinstance(e, tuple) else _xb(e) for e in res]
    return dict(groups=groups, b=b, res=res, ir=inner_relu, orr=outer_relu)


def _dense_multi(segments):
    n = segments[0]['groups'][0][0][0][0].shape[0]
    widths = [s['groups'][0][1].shape[1] for s in segments]
    co = sum(widths)
    nb = -(-n // _BN)
    ops, in_specs, plan = [], [], []

    def add_op(arr, spec):
        ops.append(arr)
        in_specs.append(spec)
        return len(ops) - 1

    def add_x(x, jb, w):
        if w == x.shape[1]:
            return add_op(x, pl.BlockSpec((_BN, w), lambda i: (i, 0)))
        if w % 128 == 0:
            return add_op(x, pl.BlockSpec((_BN, w), lambda i, j=jb: (i, j)))
        xsl = lax.slice_in_dim(x, jb * w, (jb + 1) * w, axis=1)
        return add_op(xsl, pl.BlockSpec((_BN, w), lambda i: (i, 0)))

    for s in segments:
        gplan = []
        for xs, W in s['groups']:
            xids = [add_x(x, jb, w) for x, jb, w in xs]
            wid = add_op(W, pl.BlockSpec(W.shape, lambda i: (0, 0)))
            gplan.append((xids, wid))
        rids = [add_x(x, jb, w) for x, jb, w in s['res']]
        bid = None
        if s['b'] is not None:
            wseg = s['groups'][0][1].shape[1]
            bid = add_op(s['b'].reshape(1, wseg),
                         pl.BlockSpec((1, wseg), lambda i: (0, 0)))
        plan.append((gplan, rids, bid, s['ir'], s['orr']))

    def body(*refs):
        out_ref = refs[-1]
        rs = refs[:-1]
        off = 0
        for (gplan, rids, bid, ir, orr), w in zip(plan, widths):
            acc = None
            for xids, wid in gplan:
                xsum = rs[xids[0]][...]
                for t in xids[1:]:
                    xsum = xsum + rs[t][...]
                d = jnp.dot(xsum, rs[wid][...], preferred_element_type=_F32)
                acc = d if acc is None else acc + d
            if bid is not None:
                acc = acc + rs[bid][...]
            if ir:
                acc = jnp.maximum(acc, 0.0)
            for t in rids:
                acc = acc + rs[t][...]
            if orr:
                acc = jnp.maximum(acc, 0.0)
            out_ref[:, off:off + w] = acc
            off += w

    return pl.pallas_call(
        body,
        grid=(nb,),
        in_specs=in_specs,
        out_specs=pl.BlockSpec((_BN, co), lambda i: (i, 0)),
        out_shape=jax.ShapeDtypeStruct((n, co), _F32),
    )(*ops)


def _dense(groups, b=None, res=(), inner_relu=False, outer_relu=False):
    return _dense_multi([_seg(groups, b, res, inner_relu, outer_relu)])


def _gconv(xents, ps, g, na_list, na_offs, na_ws, subWs, relu=False):
    nam = _na_multi(na_list, g)
    segs = []
    for i, p in enumerate(ps):
        groups = ([([xents[i]], p['Ws'])]
                  + _sub_range(nam, na_offs[i], na_ws[i], subWs[i]))
        segs.append(_seg(groups, b=p['b'], inner_relu=relu))
    return _dense_multi(segs)


def _eye(W):
    return jnp.eye(W.shape[1], dtype=_F32)


def _conv_blk(f, w_in, ps, g, relu=False, pre=False):
    nb = len(ps)
    xents = [_xb(f, i, w_in) for i in range(nb)]
    if pre:
        xn = _dense_multi([_seg([([xents[i]], ps[i]['Wn'])])
                           for i in range(nb)])
        ws = [p['Wn'].shape[1] for p in ps]
        offs = [sum(ws[:i]) for i in range(nb)]
        return _gconv(xents, ps, g, [xn], offs, ws,
                      [_eye(p['Wn']) for p in ps], relu=relu)
    return _gconv(xents, ps, g, [f], [i * w_in for i in range(nb)],
                  [w_in] * nb, [p['Wn'] for p in ps], relu=relu)


def _irb_blk(f, ps, g, outer_relu):
    nb = len(ps)
    c = f.shape[1] // nb
    c4, c2 = c // 4, c // 2
    A = _dense_multi(
        [_seg([([_xb(f, i, c)], p['c00']['W'])], b=p['c00']['b'],
              inner_relu=True) for i, p in enumerate(ps)]
        + [_seg([([_xb(f, i, c)], p['c10']['Wn'])])
           for i, p in enumerate(ps)])
    nam = _na_multi([A], g)
    B = _dense_multi(
        [_seg([([_xb(f, i, c)], p['c10']['Ws'])]
              + _sub_range(nam, (nb + i) * c4, c4, _eye(p['c10']['Wn'])),
              b=p['c10']['b'], inner_relu=True) for i, p in enumerate(ps)]
        + [_seg([([_xb(A, i, c4)], p['c01']['Ws'])]
                + _sub_range(nam, i * c4, c4, p['c01']['Wn']),
                b=p['c01']['b'], inner_relu=True)
           for i, p in enumerate(ps)])
    nam2 = _na_multi([B[:, :nb * c4]], g)
    segs = []
    for i, p in enumerate(ps):
        segs.append(_seg([([_xb(B, nb + i, c4)], p['c02']['W'])],
                         b=p['c02']['b'], inner_relu=True,
                         res=[_xb(f, 2 * i, c2)], outer_relu=outer_relu))
        segs.append(_seg([([_xb(B, i, c4)], p['c11']['Ws'])]
                         + _sub_range(nam2, i * c4, c4, p['c11']['Wn']),
                         b=p['c11']['b'], inner_relu=True,
                         res=[_xb(f, 2 * i + 1, c2)], outer_relu=outer_relu))
    return _dense_multi(segs)


def kernel(attr_feat, coords_feat, edge_index, up_src, edge_index_up,
           cross_src, cross_dst, edge_index_pruned, params, gt_n):
    p = params
    n = coords_feat.shape[0]
    m = up_src.shape[0]
    gt_ns = 40000

    gN = _graph(edge_index[0], edge_index[1], n)
    gU = _graph(edge_index_up[0], edge_index_up[1], m)
    gC = _graph(cross_src, cross_dst, gt_ns)
    gP = _graph(edge_index_pruned[0], edge_index_pruned[1], gt_ns)

    ps0 = [p['coords_conv0'], p['attr_conv0']]
    f = _gconv([_xb(coords_feat), _xb(attr_feat)], ps0, gN,
               [coords_feat, attr_feat], [0, 1], [1, 3],
               [q['Wn'] for q in ps0])
    f = _irb_blk(f, [p['coords_res0'], p['attr_res0']], gN, outer_relu=True)
    for li in (1, 2):
        f = _conv_blk(f, f.shape[1] // 2,
                      [p['coords_conv%d' % li], p['attr_conv%d' % li]], gN)
        f = _irb_blk(f, [p['coords_res%d' % li], p['attr_res%d' % li]], gN,
                     outer_relu=True)
    f = _conv_blk(f, f.shape[1] // 2,
                  [p['coords_conv3'], p['attr_conv3']], gN)

    f = _conv_blk(f, f.shape[1], [p['fusion0']], gN, relu=True, pre=True)
    f = _conv_blk(f, f.shape[1], [p['fusion1']], gN)
    h = f.shape[1] // 2

    cw = _dense([([_xb(f, 0, h)], p['coords_up']['W'])],
                b=p['coords_up']['b'])
    m_pad = _rup(m, _NW * _EB)
    k_up = m_pad // (_NW * _EB)
    idxr = jnp.pad(up_src, (0, m_pad - m)).reshape(_NW, k_up, _EB)
    up = _gather_fn(n, cw.shape[1], k_up)(cw, idxr)[:m]

    cn = _conv_blk(up, up.shape[1], [p['coords_convout']], gU, relu=True,
                   pre=True)
    cn = _irb_blk(cn, [p['coords_res3']], gU, outer_relu=False)
    cls = _conv_blk(cn, cn.shape[1], [p['coords_cls']], gU, pre=True)

    nam = _na_multi([f[:, h:]], gC)
    t = _dense(_sub_range(nam, 0, h, p['attr_target']['Wn']),
               b=p['attr_target']['b'])
    out = _conv_blk(t, t.shape[1], [p['attr_up_convout']], gP, relu=True,
                    pre=True)
    out = _conv_blk(out, out.shape[1], [p['conv_out']], gP, pre=True)
    out = out[:gt_ns] + (jnp.asarray(gt_n) - gt_ns).astype(out.dtype)
    return (out, cls)
